# baseline (speedup 1.0000x reference)
import jax
import jax.numpy as jnp
from jax.experimental import pallas as pl
from jax.experimental.pallas import tpu as pltpu

_B = 1024
_S = 2048
_NA = 2048
_H = 4096
_HB = 512
_SB = 256

_PREC = jax.lax.Precision.DEFAULT


def _mm1_kernel(state_ref, act_ref, w1_ref, h_ref, x_ref):
    j = pl.program_id(0)

    @pl.when(j == 0)
    def _build_x():
        x_ref[:, :_S] = state_ref[...]
        lane = jax.lax.broadcasted_iota(jnp.int32, (_B, _NA), 1)
        oh = (lane == act_ref[...]).astype(jnp.float32)
        x_ref[:, _S:] = oh

    acc = jnp.dot(x_ref[...].astype(jnp.bfloat16),
                  w1_ref[...].astype(jnp.bfloat16), precision=_PREC,
                  preferred_element_type=jnp.float32)
    h_ref[...] = jnp.maximum(acc, 0.0)


def _mm2_kernel(h_ref, w2_ref, init_ref, cols_ref, out_ref, rew_ref):
    s = pl.program_id(0)
    logits = jnp.dot(h_ref[...].astype(jnp.bfloat16),
                     w2_ref[...].astype(jnp.bfloat16), precision=_PREC,
                     preferred_element_type=jnp.float32)
    ns = jax.nn.sigmoid(logits)
    c = cols_ref[...]
    in_prog = ((c[:, 1:2] > c[:, 0:1])
               & (c[:, 2:3] >= c[:, 3:4])
               & (c[:, 2:3] >= c[:, 4:5]))
    out_ref[...] = jnp.where(in_prog, ns, init_ref[...])

    @pl.when(s == 0)
    def _reward():
        step_r = 1000.0 * (ns[:, 14:15] - ns[:, 11:12]
                           + 0.5 * (ns[:, 13:14] - ns[:, 10:11]))
        rew_ref[...] = jnp.where(in_prog, step_r, 0.0)


def kernel(initial_state, initial_action, W1, W2, Wa1, Wa2):
    act = initial_action.astype(jnp.int32).reshape(_B, 1)
    cols = initial_state[:, :8]

    h = pl.pallas_call(
        _mm1_kernel,
        grid=(_H // _HB,),
        in_specs=[
            pl.BlockSpec((_B, _S), lambda j: (0, 0)),
            pl.BlockSpec((_B, 1), lambda j: (0, 0)),
            pl.BlockSpec((_S + _NA, _HB), lambda j: (0, j)),
        ],
        out_specs=pl.BlockSpec((_B, _HB), lambda j: (0, j)),
        out_shape=jax.ShapeDtypeStruct((_B, _H), jnp.float32),
        scratch_shapes=[pltpu.VMEM((_B, _S + _NA), jnp.float32)],
    )(initial_state, act, W1)

    state_out, reward = pl.pallas_call(
        _mm2_kernel,
        grid=(_S // _SB,),
        in_specs=[
            pl.BlockSpec((_B, _H), lambda s: (0, 0)),
            pl.BlockSpec((_H, _SB), lambda s: (0, s)),
            pl.BlockSpec((_B, _SB), lambda s: (0, s)),
            pl.BlockSpec((_B, 8), lambda s: (0, 0)),
        ],
        out_specs=[
            pl.BlockSpec((_B, _SB), lambda s: (0, s)),
            pl.BlockSpec((_B, 1), lambda s: (0, 0)),
        ],
        out_shape=[
            jax.ShapeDtypeStruct((_B, _S), jnp.float32),
            jax.ShapeDtypeStruct((_B, 1), jnp.float32),
        ],
    )(h, W2, initial_state, cols)

    return jnp.concatenate([state_out, reward], axis=1)


# speedup vs baseline: 1.0218x; 1.0218x over previous
import jax
import jax.numpy as jnp
from jax.experimental import pallas as pl
from jax.experimental.pallas import tpu as pltpu

_B = 1024
_S = 2048
_NA = 2048
_H = 4096
_HB = 512
_SB = 256


def _mm1_kernel(state_ref, act_ref, w1_ref, h_ref, x_ref):
    j = pl.program_id(0)

    @pl.when(j == 0)
    def _build_x():
        x_ref[:, :_S] = state_ref[...].astype(jnp.bfloat16)
        lane = jax.lax.broadcasted_iota(jnp.int32, (_B, _NA), 1)
        oh = (lane == act_ref[...]).astype(jnp.bfloat16)
        x_ref[:, _S:] = oh

    acc = jnp.dot(x_ref[...], w1_ref[...].astype(jnp.bfloat16),
                  preferred_element_type=jnp.float32)
    h_ref[...] = jnp.maximum(acc, 0.0).astype(jnp.bfloat16)


def _mm2_kernel(h_ref, w2_ref, init_ref, cols_ref, out_ref, rew_ref):
    s = pl.program_id(0)
    logits = jnp.dot(h_ref[...], w2_ref[...].astype(jnp.bfloat16),
                     preferred_element_type=jnp.float32)
    ns = jax.nn.sigmoid(logits)
    c = cols_ref[...]
    in_prog = ((c[:, 1:2] > c[:, 0:1])
               & (c[:, 2:3] >= c[:, 3:4])
               & (c[:, 2:3] >= c[:, 4:5]))
    out_ref[...] = jnp.where(in_prog, ns, init_ref[...])

    @pl.when(s == 0)
    def _reward():
        step_r = 1000.0 * (ns[:, 14:15] - ns[:, 11:12]
                           + 0.5 * (ns[:, 13:14] - ns[:, 10:11]))
        rew_ref[...] = jnp.where(in_prog, step_r, 0.0)


def kernel(initial_state, initial_action, W1, W2, Wa1, Wa2):
    act = initial_action.astype(jnp.int32).reshape(_B, 1)
    cols = initial_state[:, :8]

    h = pl.pallas_call(
        _mm1_kernel,
        grid=(_H // _HB,),
        in_specs=[
            pl.BlockSpec((_B, _S), lambda j: (0, 0)),
            pl.BlockSpec((_B, 1), lambda j: (0, 0)),
            pl.BlockSpec((_S + _NA, _HB), lambda j: (0, j)),
        ],
        out_specs=pl.BlockSpec((_B, _HB), lambda j: (0, j)),
        out_shape=jax.ShapeDtypeStruct((_B, _H), jnp.bfloat16),
        scratch_shapes=[pltpu.VMEM((_B, _S + _NA), jnp.bfloat16)],
    )(initial_state, act, W1)

    state_out, reward = pl.pallas_call(
        _mm2_kernel,
        grid=(_S // _SB,),
        in_specs=[
            pl.BlockSpec((_B, _H), lambda s: (0, 0)),
            pl.BlockSpec((_H, _SB), lambda s: (0, s)),
            pl.BlockSpec((_B, _SB), lambda s: (0, s)),
            pl.BlockSpec((_B, 8), lambda s: (0, 0)),
        ],
        out_specs=[
            pl.BlockSpec((_B, _SB), lambda s: (0, s)),
            pl.BlockSpec((_B, 1), lambda s: (0, 0)),
        ],
        out_shape=[
            jax.ShapeDtypeStruct((_B, _S), jnp.float32),
            jax.ShapeDtypeStruct((_B, 1), jnp.float32),
        ],
    )(h, W2, initial_state, cols)

    return jnp.concatenate([state_out, reward], axis=1)


# speedup vs baseline: 1.3033x; 1.2756x over previous
import jax
import jax.numpy as jnp
from jax.experimental import pallas as pl
from jax.experimental.pallas import tpu as pltpu

_B = 1024
_S = 2048
_NA = 2048
_H = 4096
_HB = 512
_SB = 256


def _mm1_kernel(state_ref, act_ref, w1_ref, h_ref, x_ref):
    j = pl.program_id(0)

    @pl.when(j == 0)
    def _build_x():
        x_ref[:, :_S] = state_ref[...].astype(jnp.bfloat16)
        lane = jax.lax.broadcasted_iota(jnp.int32, (_B, _NA), 1)
        oh = (lane == act_ref[...]).astype(jnp.bfloat16)
        x_ref[:, _S:] = oh

    acc = jnp.dot(x_ref[...], w1_ref[...].astype(jnp.bfloat16),
                  preferred_element_type=jnp.float32)
    h_ref[...] = jnp.maximum(acc, 0.0).astype(jnp.bfloat16)


def _mm2_kernel(h_ref, w2_ref, init_ref, out_ref, mask_ref):
    s = pl.program_id(0)

    @pl.when(s == 0)
    def _mask():
        c = init_ref[...]
        in_prog = ((c[:, 1:2] > c[:, 0:1])
                   & (c[:, 2:3] >= c[:, 3:4])
                   & (c[:, 2:3] >= c[:, 4:5]))
        mask_ref[...] = in_prog

    logits = jnp.dot(h_ref[...], w2_ref[...].astype(jnp.bfloat16),
                     preferred_element_type=jnp.float32)
    ns = jax.nn.sigmoid(logits)
    in_prog = mask_ref[...]
    out_ref[:, pl.ds(s * _SB, _SB)] = jnp.where(in_prog, ns, init_ref[...])

    @pl.when(s == 0)
    def _reward():
        step_r = 1000.0 * (ns[:, 14:15] - ns[:, 11:12]
                           + 0.5 * (ns[:, 13:14] - ns[:, 10:11]))
        out_ref[:, _S:] = jnp.where(in_prog, step_r, 0.0)


def kernel(initial_state, initial_action, W1, W2, Wa1, Wa2):
    act = initial_action.astype(jnp.int32).reshape(_B, 1)

    h = pl.pallas_call(
        _mm1_kernel,
        grid=(_H // _HB,),
        in_specs=[
            pl.BlockSpec((_B, _S), lambda j: (0, 0)),
            pl.BlockSpec((_B, 1), lambda j: (0, 0)),
            pl.BlockSpec((_S + _NA, _HB), lambda j: (0, j)),
        ],
        out_specs=pl.BlockSpec((_B, _HB), lambda j: (0, j)),
        out_shape=jax.ShapeDtypeStruct((_B, _H), jnp.bfloat16),
        scratch_shapes=[pltpu.VMEM((_B, _S + _NA), jnp.bfloat16)],
    )(initial_state, act, W1)

    out = pl.pallas_call(
        _mm2_kernel,
        grid=(_S // _SB,),
        in_specs=[
            pl.BlockSpec((_B, _H), lambda s: (0, 0)),
            pl.BlockSpec((_H, _SB), lambda s: (0, s)),
            pl.BlockSpec((_B, _SB), lambda s: (0, s)),
        ],
        out_specs=pl.BlockSpec((_B, _S + 1), lambda s: (0, 0)),
        out_shape=jax.ShapeDtypeStruct((_B, _S + 1), jnp.float32),
        scratch_shapes=[pltpu.VMEM((_B, 1), jnp.bool_)],
    )(h, W2, initial_state)

    return out
